# baseline (speedup 1.0000x reference)
import functools

import jax
import jax.numpy as jnp
from jax import lax
from jax.experimental import pallas as pl
from jax.experimental.pallas import tpu as pltpu
from jax.experimental.pallas import tpu_sc as plsc

N = 10000
NP = 10240
D = 128
E = 320000
L = 16
C = 128
NC, NS = 2, 16
NW = NC * NS
EPW = E // NW
NFULL = EPW // C
TAIL = EPW - NFULL * C
TAIL_OFF = NFULL * C

GBR = 512
GBC = 1024


def _normalize(z):
    def body(z_ref, out_ref):
        zz = z_ref[...]
        norm = jnp.sqrt(jnp.sum(zz * zz, axis=1, keepdims=True))
        out_ref[...] = zz / jnp.maximum(norm, 1e-12)

    return pl.pallas_call(
        body,
        out_shape=jax.ShapeDtypeStruct((N, D), jnp.float32),
        grid=(10,),
        in_specs=[pl.BlockSpec((N // 10, D), lambda i: (i, 0))],
        out_specs=pl.BlockSpec((N // 10, D), lambda i: (i, 0)),
    )(z)


def _gram(zn):
    def body(a_ref, b_ref, out_ref):
        out_ref[...] = lax.dot_general(
            a_ref[...], b_ref[...],
            dimension_numbers=(((1,), (1,)), ((), ())),
            preferred_element_type=jnp.float32,
        )

    return pl.pallas_call(
        body,
        out_shape=jax.ShapeDtypeStruct((NP, NP), jnp.float32),
        grid=(NP // GBR, NP // GBC),
        in_specs=[
            pl.BlockSpec((GBR, D), lambda i, j: (i, 0)),
            pl.BlockSpec((GBC, D), lambda i, j: (j, 0)),
        ],
        out_specs=pl.BlockSpec((GBR, GBC), lambda i, j: (i, j)),
    )(zn, zn)


def _make_sc_kernel():
    mesh = plsc.VectorSubcoreMesh(core_axis_name="c", subcore_axis_name="s")

    @functools.partial(
        pl.kernel,
        out_type=jax.ShapeDtypeStruct((E,), jnp.float32),
        mesh=mesh,
        compiler_params=pltpu.CompilerParams(needs_layout_passes=False),
        scratch_types=[
            pltpu.VMEM((EPW,), jnp.int32),
            pltpu.VMEM((EPW,), jnp.int32),
            pltpu.VMEM((EPW,), jnp.int32),
            pltpu.VMEM((EPW,), jnp.float32),
            pltpu.SemaphoreType.DMA,
            pltpu.SemaphoreType.DMA,
        ],
    )
    def scorer(g_hbm, src_hbm, tgt_hbm, out_hbm, sidx, tidx, fidx, scores,
               sem0, sem1):
        w = lax.axis_index("s") * NC + lax.axis_index("c")
        base = w * EPW
        sems = (sem0, sem1)

        pltpu.sync_copy(src_hbm.at[pl.ds(base, EPW)], sidx)
        pltpu.sync_copy(tgt_hbm.at[pl.ds(base, EPW)], tidx)

        def flatten(v, _):
            off = pl.multiple_of(v * L, L)
            s = sidx[pl.ds(off, L)]
            t = tidx[pl.ds(off, L)]
            fidx[pl.ds(off, L)] = s * NP + t
            return _

        lax.fori_loop(0, EPW // L, flatten, None)

        def start(i, b, n):
            off = pl.multiple_of(i * C, C)
            pltpu.async_copy(g_hbm.at[fidx.at[pl.ds(off, n)]],
                             scores.at[pl.ds(off, n)], sems[b])

        def wait(b, n):
            pltpu.make_async_copy(g_hbm.at[pl.ds(0, n)],
                                  scores.at[pl.ds(0, n)], sems[b]).wait()

        def compute(i, n):
            coff = pl.multiple_of(i * C, C)
            for g in range(n // L):
                v = scores[pl.ds(coff + g * L, L)]
                scores[pl.ds(coff + g * L, L)] = jnp.exp(v - 1.0)

        start(0, 0, C)
        start(1, 1, C)

        def pair(t, carry):
            for b in range(2):
                i = 2 * t + b
                wait(b, C)
                compute(i, C)

                @pl.when(i + 2 < NFULL)
                def _():
                    start(i + 2, b, C)

            return carry

        lax.fori_loop(0, NFULL // 2, pair, None)

        pltpu.sync_copy(g_hbm.at[fidx.at[pl.ds(TAIL_OFF, TAIL)]],
                        scores.at[pl.ds(TAIL_OFF, TAIL)])
        compute(NFULL, TAIL)

        pltpu.sync_copy(scores, out_hbm.at[pl.ds(base, EPW)])

    return scorer


_sc_score = _make_sc_kernel()


def kernel(z, edge_index):
    zn = _normalize(z.astype(jnp.float32))
    g = _gram(zn).reshape((NP * NP,))
    ei = edge_index.astype(jnp.int32)
    return _sc_score(g, ei[0], ei[1])


# speedup vs baseline: 2.6352x; 2.6352x over previous
import functools

import jax
import jax.numpy as jnp
from jax import lax
from jax.experimental import pallas as pl
from jax.experimental.pallas import tpu as pltpu
from jax.experimental.pallas import tpu_sc as plsc

N = 10000
NP = 10240
D = 128
E = 320000
L = 16
C = 128
NC, NS = 2, 16
NW = NC * NS
EPW = E // NW
NFULL = EPW // C
TAIL = EPW - NFULL * C
TAIL_OFF = NFULL * C

GBR = 128


def _normalize(z):
    def body(z_ref, out_ref):
        zz = z_ref[...]
        norm = jnp.sqrt(jnp.sum(zz * zz, axis=1, keepdims=True))
        out_ref[...] = zz / jnp.maximum(norm, 1e-12)

    return pl.pallas_call(
        body,
        out_shape=jax.ShapeDtypeStruct((N, D), jnp.float32),
        grid=(10,),
        in_specs=[pl.BlockSpec((N // 10, D), lambda i: (i, 0))],
        out_specs=pl.BlockSpec((N // 10, D), lambda i: (i, 0)),
    )(z)


def _gram_flat(znp):
    def body(a_ref, b_ref, out_ref):
        res = lax.dot_general(
            a_ref[...], b_ref[...],
            dimension_numbers=(((1,), (1,)), ((), ())),
            preferred_element_type=jnp.float32,
        )
        out_ref[...] = res.reshape(GBR * NP)

    return pl.pallas_call(
        body,
        out_shape=jax.ShapeDtypeStruct((NP * NP,), jnp.float32),
        grid=(NP // GBR,),
        in_specs=[
            pl.BlockSpec((GBR, D), lambda i: (i, 0)),
            pl.BlockSpec((NP, D), lambda i: (0, 0)),
        ],
        out_specs=pl.BlockSpec((GBR * NP,), lambda i: (i,)),
    )(znp, znp)


def _make_sc_kernel():
    mesh = plsc.VectorSubcoreMesh(core_axis_name="c", subcore_axis_name="s")

    @functools.partial(
        pl.kernel,
        out_type=jax.ShapeDtypeStruct((E,), jnp.float32),
        mesh=mesh,
        compiler_params=pltpu.CompilerParams(needs_layout_passes=False),
        scratch_types=[
            pltpu.VMEM((EPW,), jnp.int32),
            pltpu.VMEM((EPW,), jnp.int32),
            pltpu.VMEM((EPW,), jnp.int32),
            pltpu.VMEM((EPW,), jnp.float32),
            pltpu.SemaphoreType.DMA,
            pltpu.SemaphoreType.DMA,
        ],
    )
    def scorer(g_hbm, src_hbm, tgt_hbm, out_hbm, sidx, tidx, fidx, scores,
               sem0, sem1):
        w = lax.axis_index("s") * NC + lax.axis_index("c")
        base = w * EPW
        sems = (sem0, sem1)

        pltpu.sync_copy(src_hbm.at[pl.ds(base, EPW)], sidx)
        pltpu.sync_copy(tgt_hbm.at[pl.ds(base, EPW)], tidx)

        def flatten(v, _):
            off = pl.multiple_of(v * L, L)
            s = sidx[pl.ds(off, L)]
            t = tidx[pl.ds(off, L)]
            fidx[pl.ds(off, L)] = s * NP + t
            return _

        lax.fori_loop(0, EPW // L, flatten, None)

        def start(i, b, n):
            off = pl.multiple_of(i * C, C)
            pltpu.async_copy(g_hbm.at[fidx.at[pl.ds(off, n)]],
                             scores.at[pl.ds(off, n)], sems[b])

        def wait(b, n):
            pltpu.make_async_copy(g_hbm.at[pl.ds(0, n)],
                                  scores.at[pl.ds(0, n)], sems[b]).wait()

        def compute(i, n):
            coff = pl.multiple_of(i * C, C)
            for g in range(n // L):
                v = scores[pl.ds(coff + g * L, L)]
                scores[pl.ds(coff + g * L, L)] = jnp.exp(v - 1.0)

        start(0, 0, C)
        start(1, 1, C)

        def pair(t, carry):
            for b in range(2):
                i = 2 * t + b
                wait(b, C)
                compute(i, C)

                @pl.when(i + 2 < NFULL)
                def _():
                    start(i + 2, b, C)

            return carry

        lax.fori_loop(0, NFULL // 2, pair, None)

        pltpu.sync_copy(g_hbm.at[fidx.at[pl.ds(TAIL_OFF, TAIL)]],
                        scores.at[pl.ds(TAIL_OFF, TAIL)])
        compute(NFULL, TAIL)

        pltpu.sync_copy(scores, out_hbm.at[pl.ds(base, EPW)])

    return scorer


_sc_score = _make_sc_kernel()


def kernel(z, edge_index):
    zn = _normalize(z.astype(jnp.float32))
    znp = jnp.pad(zn, ((0, NP - N), (0, 0)))
    g = _gram_flat(znp)
    ei = edge_index.astype(jnp.int32)
    return _sc_score(g, ei[0], ei[1])


# speedup vs baseline: 2.6981x; 1.0238x over previous
import functools

import jax
import jax.numpy as jnp
from jax import lax
from jax.experimental import pallas as pl
from jax.experimental.pallas import tpu as pltpu
from jax.experimental.pallas import tpu_sc as plsc

N = 10000
NP = 10240
D = 128
E = 320000
L = 16
C = 128
NC, NS = 2, 16
NW = NC * NS
EPW = E // NW
NFULL = EPW // C
TAIL = EPW - NFULL * C
TAIL_OFF = NFULL * C

GBR = 128


def _normalize(z):
    def body(z_ref, out_ref):
        zz = z_ref[...]
        norm = jnp.sqrt(jnp.sum(zz * zz, axis=1, keepdims=True))
        out_ref[...] = (zz / jnp.maximum(norm, 1e-12)).astype(jnp.bfloat16)

    return pl.pallas_call(
        body,
        out_shape=jax.ShapeDtypeStruct((N, D), jnp.bfloat16),
        grid=(10,),
        in_specs=[pl.BlockSpec((N // 10, D), lambda i: (i, 0))],
        out_specs=pl.BlockSpec((N // 10, D), lambda i: (i, 0)),
    )(z)


def _gram_flat(znp):
    def body(a_ref, b_ref, out_ref):
        res = lax.dot_general(
            a_ref[...], b_ref[...],
            dimension_numbers=(((1,), (1,)), ((), ())),
            preferred_element_type=jnp.float32,
        )
        out_ref[...] = res.reshape(GBR * NP)

    return pl.pallas_call(
        body,
        out_shape=jax.ShapeDtypeStruct((NP * NP,), jnp.float32),
        grid=(NP // GBR,),
        in_specs=[
            pl.BlockSpec((GBR, D), lambda i: (i, 0)),
            pl.BlockSpec((NP, D), lambda i: (0, 0)),
        ],
        out_specs=pl.BlockSpec((GBR * NP,), lambda i: (i,)),
    )(znp, znp)


def _make_sc_kernel():
    mesh = plsc.VectorSubcoreMesh(core_axis_name="c", subcore_axis_name="s")

    @functools.partial(
        pl.kernel,
        out_type=jax.ShapeDtypeStruct((E,), jnp.float32),
        mesh=mesh,
        compiler_params=pltpu.CompilerParams(needs_layout_passes=False),
        scratch_types=[
            pltpu.VMEM((EPW,), jnp.int32),
            pltpu.VMEM((EPW,), jnp.int32),
            pltpu.VMEM((EPW,), jnp.int32),
            pltpu.VMEM((EPW,), jnp.float32),
            pltpu.SemaphoreType.DMA,
            pltpu.SemaphoreType.DMA,
        ],
    )
    def scorer(g_hbm, src_hbm, tgt_hbm, out_hbm, sidx, tidx, fidx, scores,
               sem0, sem1):
        w = lax.axis_index("s") * NC + lax.axis_index("c")
        base = w * EPW
        sems = (sem0, sem1)

        pltpu.sync_copy(src_hbm.at[pl.ds(base, EPW)], sidx)
        pltpu.sync_copy(tgt_hbm.at[pl.ds(base, EPW)], tidx)

        def flatten(v, _):
            off = pl.multiple_of(v * L, L)
            s = sidx[pl.ds(off, L)]
            t = tidx[pl.ds(off, L)]
            fidx[pl.ds(off, L)] = s * NP + t
            return _

        lax.fori_loop(0, EPW // L, flatten, None)

        def start(i, b, n):
            off = pl.multiple_of(i * C, C)
            pltpu.async_copy(g_hbm.at[fidx.at[pl.ds(off, n)]],
                             scores.at[pl.ds(off, n)], sems[b])

        def wait(b, n):
            pltpu.make_async_copy(g_hbm.at[pl.ds(0, n)],
                                  scores.at[pl.ds(0, n)], sems[b]).wait()

        def compute(i, n):
            coff = pl.multiple_of(i * C, C)
            for g in range(n // L):
                v = scores[pl.ds(coff + g * L, L)]
                scores[pl.ds(coff + g * L, L)] = jnp.exp(v - 1.0)

        start(0, 0, C)
        start(1, 1, C)

        def pair(t, carry):
            for b in range(2):
                i = 2 * t + b
                wait(b, C)
                compute(i, C)

                @pl.when(i + 2 < NFULL)
                def _():
                    start(i + 2, b, C)

            return carry

        lax.fori_loop(0, NFULL // 2, pair, None)

        pltpu.sync_copy(g_hbm.at[fidx.at[pl.ds(TAIL_OFF, TAIL)]],
                        scores.at[pl.ds(TAIL_OFF, TAIL)])
        compute(NFULL, TAIL)

        pltpu.sync_copy(scores, out_hbm.at[pl.ds(base, EPW)])

    return scorer


_sc_score = _make_sc_kernel()


def kernel(z, edge_index):
    zn = _normalize(z.astype(jnp.float32))
    znp = jnp.pad(zn, ((0, NP - N), (0, 0)))
    g = _gram_flat(znp)
    ei = edge_index.astype(jnp.int32)
    return _sc_score(g, ei[0], ei[1])


# speedup vs baseline: 2.9964x; 1.1106x over previous
import functools

import jax
import jax.numpy as jnp
from jax import lax
from jax.experimental import pallas as pl
from jax.experimental.pallas import tpu as pltpu
from jax.experimental.pallas import tpu_sc as plsc

N = 10000
NP = 10240
D = 128
E = 320000
L = 16
C = 128
NC, NS = 2, 16
NW = NC * NS
EPW = E // NW
NFULL = EPW // C
TAIL = EPW - NFULL * C
TAIL_OFF = NFULL * C

GBR = 128


def _normalize(z):
    def body(z_ref, out_ref):
        zz = z_ref[...]
        norm = jnp.sqrt(jnp.sum(zz * zz, axis=1, keepdims=True))
        out_ref[...] = (zz / jnp.maximum(norm, 1e-12)).astype(jnp.bfloat16)

    return pl.pallas_call(
        body,
        out_shape=jax.ShapeDtypeStruct((N, D), jnp.bfloat16),
        grid=(10,),
        in_specs=[pl.BlockSpec((N // 10, D), lambda i: (i, 0))],
        out_specs=pl.BlockSpec((N // 10, D), lambda i: (i, 0)),
    )(z)


def _gram_flat(znp):
    def body(a_ref, b_ref, out_ref):
        res = lax.dot_general(
            a_ref[...], b_ref[...],
            dimension_numbers=(((1,), (1,)), ((), ())),
            preferred_element_type=jnp.float32,
        )
        lo = lax.bitcast_convert_type(
            res[:, :NP // 2].astype(jnp.bfloat16), jnp.uint16
        ).astype(jnp.uint32)
        hi = lax.bitcast_convert_type(
            res[:, NP // 2:].astype(jnp.bfloat16), jnp.uint16
        ).astype(jnp.uint32)
        words = lax.bitcast_convert_type(lo | (hi << 16), jnp.int32)
        out_ref[...] = words.reshape(GBR * NP // 2)

    return pl.pallas_call(
        body,
        out_shape=jax.ShapeDtypeStruct((NP * NP // 2,), jnp.int32),
        grid=(NP // GBR,),
        in_specs=[
            pl.BlockSpec((GBR, D), lambda i: (i, 0)),
            pl.BlockSpec((NP, D), lambda i: (0, 0)),
        ],
        out_specs=pl.BlockSpec((GBR * NP // 2,), lambda i: (i,)),
    )(znp, znp)


def _make_sc_kernel():
    mesh = plsc.VectorSubcoreMesh(core_axis_name="c", subcore_axis_name="s")

    @functools.partial(
        pl.kernel,
        out_type=jax.ShapeDtypeStruct((E,), jnp.float32),
        mesh=mesh,
        compiler_params=pltpu.CompilerParams(needs_layout_passes=False),
        scratch_types=[
            pltpu.VMEM((EPW,), jnp.int32),
            pltpu.VMEM((EPW,), jnp.int32),
            pltpu.VMEM((EPW,), jnp.int32),
            pltpu.VMEM((EPW,), jnp.int32),
            pltpu.VMEM((EPW,), jnp.float32),
            pltpu.SemaphoreType.DMA,
            pltpu.SemaphoreType.DMA,
        ],
    )
    def scorer(g_hbm, src_hbm, tgt_hbm, out_hbm, sidx, tidx, fidx, words,
               scores, sem0, sem1):
        w = lax.axis_index("s") * NC + lax.axis_index("c")
        base = w * EPW
        sems = (sem0, sem1)

        pltpu.sync_copy(src_hbm.at[pl.ds(base, EPW)], sidx)
        pltpu.sync_copy(tgt_hbm.at[pl.ds(base, EPW)], tidx)

        def flatten(v, _):
            off = pl.multiple_of(v * L, L)
            s = sidx[pl.ds(off, L)]
            t = tidx[pl.ds(off, L)]
            tm = jnp.where(t >= NP // 2, t - NP // 2, t)
            fidx[pl.ds(off, L)] = s * (NP // 2) + tm
            return _

        lax.fori_loop(0, EPW // L, flatten, None)

        def start(i, b, n):
            off = pl.multiple_of(i * C, C)
            pltpu.async_copy(g_hbm.at[fidx.at[pl.ds(off, n)]],
                             words.at[pl.ds(off, n)], sems[b])

        def wait(b, n):
            pltpu.make_async_copy(g_hbm.at[pl.ds(0, n)],
                                  words.at[pl.ds(0, n)], sems[b]).wait()

        def compute(i, n):
            coff = pl.multiple_of(i * C, C)
            for g in range(n // L):
                off = pl.multiple_of(coff + g * L, L)
                w = words[pl.ds(off, L)]
                in_hi = tidx[pl.ds(off, L)] >= NP // 2
                bits = jnp.where(in_hi, w & jnp.int32(-65536), w << 16)
                v = lax.bitcast_convert_type(bits, jnp.float32)
                scores[pl.ds(off, L)] = jnp.exp(v - 1.0)

        start(0, 0, C)
        start(1, 1, C)

        def pair(t, carry):
            for b in range(2):
                i = 2 * t + b
                wait(b, C)
                compute(i, C)

                @pl.when(i + 2 < NFULL)
                def _():
                    start(i + 2, b, C)

            return carry

        lax.fori_loop(0, NFULL // 2, pair, None)

        pltpu.sync_copy(g_hbm.at[fidx.at[pl.ds(TAIL_OFF, TAIL)]],
                        words.at[pl.ds(TAIL_OFF, TAIL)])
        compute(NFULL, TAIL)

        pltpu.sync_copy(scores, out_hbm.at[pl.ds(base, EPW)])

    return scorer


_sc_score = _make_sc_kernel()


def kernel(z, edge_index):
    zn = _normalize(z.astype(jnp.float32))
    znp = jnp.pad(zn, ((0, NP - N), (0, 0)))
    g = _gram_flat(znp)
    ei = edge_index.astype(jnp.int32)
    return _sc_score(g, ei[0], ei[1])


# speedup vs baseline: 3.0717x; 1.0251x over previous
import functools

import jax
import jax.numpy as jnp
from jax import lax
from jax.experimental import pallas as pl
from jax.experimental.pallas import tpu as pltpu
from jax.experimental.pallas import tpu_sc as plsc

N = 10000
NP = 10240
D = 128
E = 320000
L = 16
C = 128
NC, NS = 2, 16
NW = NC * NS
EPW = E // NW
NFULL = EPW // C
TAIL = EPW - NFULL * C
TAIL_OFF = NFULL * C

GBR = 512


def _normalize(z):
    def body(z_ref, out_ref):
        zz = z_ref[...]
        norm = jnp.sqrt(jnp.sum(zz * zz, axis=1, keepdims=True))
        out_ref[...] = (zz / jnp.maximum(norm, 1e-12)).astype(jnp.bfloat16)

    return pl.pallas_call(
        body,
        out_shape=jax.ShapeDtypeStruct((N, D), jnp.bfloat16),
        grid=(10,),
        in_specs=[pl.BlockSpec((N // 10, D), lambda i: (i, 0))],
        out_specs=pl.BlockSpec((N // 10, D), lambda i: (i, 0)),
    )(z)


def _gram_flat(znp):
    def body(a_ref, b_ref, out_ref):
        res = lax.dot_general(
            a_ref[...], b_ref[...],
            dimension_numbers=(((1,), (1,)), ((), ())),
            preferred_element_type=jnp.float32,
        )
        lo = lax.bitcast_convert_type(
            res[:, :NP // 2].astype(jnp.bfloat16), jnp.uint16
        ).astype(jnp.uint32)
        hi = lax.bitcast_convert_type(
            res[:, NP // 2:].astype(jnp.bfloat16), jnp.uint16
        ).astype(jnp.uint32)
        words = lax.bitcast_convert_type(lo | (hi << 16), jnp.int32)
        out_ref[...] = words.reshape(GBR * NP // 2)

    return pl.pallas_call(
        body,
        out_shape=jax.ShapeDtypeStruct((NP * NP // 2,), jnp.int32),
        grid=(NP // GBR,),
        in_specs=[
            pl.BlockSpec((GBR, D), lambda i: (i, 0)),
            pl.BlockSpec((NP, D), lambda i: (0, 0)),
        ],
        out_specs=pl.BlockSpec((GBR * NP // 2,), lambda i: (i,)),
    )(znp, znp)


def _make_sc_kernel():
    mesh = plsc.VectorSubcoreMesh(core_axis_name="c", subcore_axis_name="s")

    @functools.partial(
        pl.kernel,
        out_type=jax.ShapeDtypeStruct((E,), jnp.float32),
        mesh=mesh,
        compiler_params=pltpu.CompilerParams(needs_layout_passes=False),
        scratch_types=[
            pltpu.VMEM((EPW,), jnp.int32),
            pltpu.VMEM((EPW,), jnp.int32),
            pltpu.VMEM((EPW,), jnp.int32),
            pltpu.VMEM((EPW,), jnp.int32),
            pltpu.VMEM((EPW,), jnp.float32),
            pltpu.SemaphoreType.DMA,
            pltpu.SemaphoreType.DMA,
        ],
    )
    def scorer(g_hbm, src_hbm, tgt_hbm, out_hbm, sidx, tidx, fidx, words,
               scores, sem0, sem1):
        w = lax.axis_index("s") * NC + lax.axis_index("c")
        base = w * EPW
        sems = (sem0, sem1)

        pltpu.sync_copy(src_hbm.at[pl.ds(base, EPW)], sidx)
        pltpu.sync_copy(tgt_hbm.at[pl.ds(base, EPW)], tidx)

        def flatten(v, _):
            off = pl.multiple_of(v * L, L)
            s = sidx[pl.ds(off, L)]
            t = tidx[pl.ds(off, L)]
            tm = jnp.where(t >= NP // 2, t - NP // 2, t)
            fidx[pl.ds(off, L)] = s * (NP // 2) + tm
            return _

        lax.fori_loop(0, EPW // L, flatten, None)

        def start(i, b, n):
            off = pl.multiple_of(i * C, C)
            pltpu.async_copy(g_hbm.at[fidx.at[pl.ds(off, n)]],
                             words.at[pl.ds(off, n)], sems[b])

        def wait(b, n):
            pltpu.make_async_copy(g_hbm.at[pl.ds(0, n)],
                                  words.at[pl.ds(0, n)], sems[b]).wait()

        def compute(i, n):
            coff = pl.multiple_of(i * C, C)
            for g in range(n // L):
                off = pl.multiple_of(coff + g * L, L)
                w = words[pl.ds(off, L)]
                in_hi = tidx[pl.ds(off, L)] >= NP // 2
                bits = jnp.where(in_hi, w & jnp.int32(-65536), w << 16)
                v = lax.bitcast_convert_type(bits, jnp.float32)
                scores[pl.ds(off, L)] = jnp.exp(v - 1.0)

        start(0, 0, C)
        start(1, 1, C)

        def pair(t, carry):
            for b in range(2):
                i = 2 * t + b
                wait(b, C)
                compute(i, C)

                @pl.when(i + 2 < NFULL)
                def _():
                    start(i + 2, b, C)

            return carry

        lax.fori_loop(0, NFULL // 2, pair, None)

        pltpu.sync_copy(g_hbm.at[fidx.at[pl.ds(TAIL_OFF, TAIL)]],
                        words.at[pl.ds(TAIL_OFF, TAIL)])
        compute(NFULL, TAIL)

        pltpu.sync_copy(scores, out_hbm.at[pl.ds(base, EPW)])

    return scorer


_sc_score = _make_sc_kernel()


def kernel(z, edge_index):
    zn = _normalize(z.astype(jnp.float32))
    znp = jnp.pad(zn, ((0, NP - N), (0, 0)))
    g = _gram_flat(znp)
    ei = edge_index.astype(jnp.int32)
    return _sc_score(g, ei[0], ei[1])
